# baseline (speedup 1.0000x reference)
import jax
import jax.numpy as jnp
from jax.experimental import pallas as pl
from jax.experimental.pallas import tpu as pltpu

VOCAB = 256
EMBED = 256
HIDDEN = 512
SEQ = 256
BATCH = 64


def _lstm_kernel(xs_ref, emb_ref, wih_ref, whh_ref, bias_ref,
                 wfc_ref, bfc_ref, out_ref, table_ref):
    table_f32 = jnp.dot(emb_ref[...], wih_ref[...],
                        preferred_element_type=jnp.float32) + bias_ref[...]
    table_ref[...] = table_f32.astype(jnp.bfloat16)

    HB = BATCH // 2

    def half_step(xt, h, c):
        onehot = (xt[:, None] == jax.lax.broadcasted_iota(
            jnp.int32, (HB, VOCAB), 1)).astype(jnp.bfloat16)
        inp = jnp.dot(onehot, table_ref[...],
                      preferred_element_type=jnp.float32)
        gates = inp + jnp.dot(h.astype(jnp.bfloat16), whh_ref[...],
                              preferred_element_type=jnp.float32)
        t4 = jnp.tanh(gates)
        i = 0.5 * t4[:, 0 * HIDDEN:1 * HIDDEN] + 0.5
        f = 0.5 * t4[:, 1 * HIDDEN:2 * HIDDEN] + 0.5
        g = t4[:, 2 * HIDDEN:3 * HIDDEN]
        o = 0.5 * t4[:, 3 * HIDDEN:4 * HIDDEN] + 0.5
        c_new = f * c + i * g
        h_new = o * jnp.tanh(c_new)
        return h_new, c_new

    def step(t, carry):
        ha, ca, hb, cb = carry
        xt = xs_ref[t, 0, :]
        ha, ca = half_step(xt[:HB], ha, ca)
        hb, cb = half_step(xt[HB:], hb, cb)
        return ha, ca, hb, cb

    h0 = jnp.zeros((HB, HIDDEN), dtype=jnp.float32)
    c0 = jnp.zeros((HB, HIDDEN), dtype=jnp.float32)
    ha, _, hb, _ = jax.lax.fori_loop(0, SEQ, step, (h0, c0, h0, c0))
    h_last = jnp.concatenate([ha, hb], axis=0)
    out_ref[...] = jnp.dot(h_last, wfc_ref[...],
                           preferred_element_type=jnp.float32) + bfc_ref[...]


def kernel(x, emb, W_ih, W_hh, b_ih, b_hh, W_fc, b_fc):
    xs = jnp.transpose(x.astype(jnp.int32), (1, 0)).reshape(SEQ, 1, BATCH)
    col = jax.lax.broadcasted_iota(jnp.int32, (1, 4 * HIDDEN), 1)
    scale = jnp.where((col >= 2 * HIDDEN) & (col < 3 * HIDDEN), 1.0, 0.5)
    wih_t = W_ih.T * scale
    whh_t = (W_hh.T * scale).astype(jnp.bfloat16)
    bias = (b_ih + b_hh).reshape(1, 4 * HIDDEN) * scale
    wfc_t = W_fc.T
    bfc = b_fc.reshape(1, VOCAB)

    return pl.pallas_call(
        _lstm_kernel,
        out_shape=jax.ShapeDtypeStruct((BATCH, VOCAB), jnp.float32),
        scratch_shapes=[
            pltpu.VMEM((VOCAB, 4 * HIDDEN), jnp.bfloat16),
        ],
    )(xs, emb, wih_t, whh_t, bias, wfc_t, bfc)


# speedup vs baseline: 1.8700x; 1.8700x over previous
import jax
import jax.numpy as jnp
from jax.experimental import pallas as pl
from jax.experimental.pallas import tpu as pltpu

VOCAB = 256
EMBED = 256
HIDDEN = 512
SEQ = 256
BATCH = 64
G = 8
GROUP_ROWS = G * BATCH


def _lstm_kernel(xs_ref, emb_ref, wih_ref, whh_ref, bias_ref,
                 wfc_ref, bfc_ref, out_ref, table_ref, buf_ref):
    table_f32 = jnp.dot(emb_ref[...], wih_ref[...],
                        preferred_element_type=jnp.float32) + bias_ref[...]
    table_ref[...] = table_f32.astype(jnp.bfloat16)

    def lstm_step(inp, h, c):
        gates = inp + jnp.dot(h.astype(jnp.bfloat16), whh_ref[...],
                              preferred_element_type=jnp.float32)
        t4 = jnp.tanh(gates)
        i = 0.5 * t4[:, 0 * HIDDEN:1 * HIDDEN] + 0.5
        f = 0.5 * t4[:, 1 * HIDDEN:2 * HIDDEN] + 0.5
        g = t4[:, 2 * HIDDEN:3 * HIDDEN]
        o = 0.5 * t4[:, 3 * HIDDEN:4 * HIDDEN] + 0.5
        c_new = f * c + i * g
        h_new = o * jnp.tanh(c_new)
        return h_new, c_new

    def group(gi, carry):
        h, c = carry
        xg = xs_ref[pl.ds(gi * GROUP_ROWS, GROUP_ROWS), :]
        onehot = (xg == jax.lax.broadcasted_iota(
            jnp.int32, (GROUP_ROWS, VOCAB), 1)).astype(jnp.bfloat16)
        inp_g = jnp.dot(onehot, table_ref[...],
                        preferred_element_type=jnp.float32)
        buf_ref[...] = inp_g.astype(jnp.bfloat16)
        for k in range(G):
            inp = buf_ref[k * BATCH:(k + 1) * BATCH, :].astype(jnp.float32)
            h, c = lstm_step(inp, h, c)
        return h, c

    h0 = jnp.zeros((BATCH, HIDDEN), dtype=jnp.float32)
    c0 = jnp.zeros((BATCH, HIDDEN), dtype=jnp.float32)
    h_last, _ = jax.lax.fori_loop(0, SEQ // G, group, (h0, c0))
    out_ref[...] = jnp.dot(h_last, wfc_ref[...],
                           preferred_element_type=jnp.float32) + bfc_ref[...]


def kernel(x, emb, W_ih, W_hh, b_ih, b_hh, W_fc, b_fc):
    xs = jnp.transpose(x.astype(jnp.int32), (1, 0)).reshape(SEQ * BATCH, 1)
    col = jax.lax.broadcasted_iota(jnp.int32, (1, 4 * HIDDEN), 1)
    scale = jnp.where((col >= 2 * HIDDEN) & (col < 3 * HIDDEN), 1.0, 0.5)
    wih_t = W_ih.T * scale
    whh_t = (W_hh.T * scale).astype(jnp.bfloat16)
    bias = (b_ih + b_hh).reshape(1, 4 * HIDDEN) * scale
    wfc_t = W_fc.T
    bfc = b_fc.reshape(1, VOCAB)

    return pl.pallas_call(
        _lstm_kernel,
        out_shape=jax.ShapeDtypeStruct((BATCH, VOCAB), jnp.float32),
        scratch_shapes=[
            pltpu.VMEM((VOCAB, 4 * HIDDEN), jnp.bfloat16),
            pltpu.VMEM((GROUP_ROWS, 4 * HIDDEN), jnp.bfloat16),
        ],
    )(xs, emb, wih_t, whh_t, bias, wfc_t, bfc)


# speedup vs baseline: 1.8927x; 1.0121x over previous
import jax
import jax.numpy as jnp
from jax.experimental import pallas as pl
from jax.experimental.pallas import tpu as pltpu

VOCAB = 256
EMBED = 256
HIDDEN = 512
SEQ = 256
BATCH = 64
G = 16
GROUP_ROWS = G * BATCH


def _lstm_kernel(xs_ref, emb_ref, wih_ref, whh_ref, bias_ref,
                 wfc_ref, bfc_ref, out_ref, table_ref, buf_ref):
    table_f32 = jnp.dot(emb_ref[...], wih_ref[...],
                        preferred_element_type=jnp.float32) + bias_ref[...]
    table_ref[...] = table_f32.astype(jnp.bfloat16)

    def lstm_step(inp, h, c):
        gates = inp + jnp.dot(h.astype(jnp.bfloat16), whh_ref[...],
                              preferred_element_type=jnp.float32)
        t4 = jnp.tanh(gates)
        i = 0.5 * t4[:, 0 * HIDDEN:1 * HIDDEN] + 0.5
        f = 0.5 * t4[:, 1 * HIDDEN:2 * HIDDEN] + 0.5
        g = t4[:, 2 * HIDDEN:3 * HIDDEN]
        o = 0.5 * t4[:, 3 * HIDDEN:4 * HIDDEN] + 0.5
        c_new = f * c + i * g
        h_new = o * jnp.tanh(c_new)
        return h_new, c_new

    def group(gi, carry):
        h, c = carry
        xg = xs_ref[pl.ds(gi * GROUP_ROWS, GROUP_ROWS), :]
        onehot = (xg == jax.lax.broadcasted_iota(
            jnp.int32, (GROUP_ROWS, VOCAB), 1)).astype(jnp.bfloat16)
        inp_g = jnp.dot(onehot, table_ref[...],
                        preferred_element_type=jnp.float32)
        buf_ref[...] = inp_g.astype(jnp.bfloat16)
        for k in range(G):
            inp = buf_ref[k * BATCH:(k + 1) * BATCH, :].astype(jnp.float32)
            h, c = lstm_step(inp, h, c)
        return h, c

    h0 = jnp.zeros((BATCH, HIDDEN), dtype=jnp.float32)
    c0 = jnp.zeros((BATCH, HIDDEN), dtype=jnp.float32)
    h_last, _ = jax.lax.fori_loop(0, SEQ // G, group, (h0, c0))
    out_ref[...] = jnp.dot(h_last, wfc_ref[...],
                           preferred_element_type=jnp.float32) + bfc_ref[...]


def kernel(x, emb, W_ih, W_hh, b_ih, b_hh, W_fc, b_fc):
    xs = jnp.transpose(x.astype(jnp.int32), (1, 0)).reshape(SEQ * BATCH, 1)
    col = jax.lax.broadcasted_iota(jnp.int32, (1, 4 * HIDDEN), 1)
    scale = jnp.where((col >= 2 * HIDDEN) & (col < 3 * HIDDEN), 1.0, 0.5)
    wih_t = W_ih.T * scale
    whh_t = (W_hh.T * scale).astype(jnp.bfloat16)
    bias = (b_ih + b_hh).reshape(1, 4 * HIDDEN) * scale
    wfc_t = W_fc.T
    bfc = b_fc.reshape(1, VOCAB)

    return pl.pallas_call(
        _lstm_kernel,
        out_shape=jax.ShapeDtypeStruct((BATCH, VOCAB), jnp.float32),
        scratch_shapes=[
            pltpu.VMEM((VOCAB, 4 * HIDDEN), jnp.bfloat16),
            pltpu.VMEM((GROUP_ROWS, 4 * HIDDEN), jnp.bfloat16),
        ],
    )(xs, emb, wih_t, whh_t, bias, wfc_t, bfc)
